# Initial kernel scaffold; baseline (speedup 1.0000x reference)
import jax
import jax.numpy as jnp
from jax.experimental import pallas as pl


def kernel(src, w, b, gamma, beta):
    raise NotImplementedError("write your pallas kernel here")


# speedup vs baseline: 1.3181x; 1.3181x over previous
import jax
import jax.numpy as jnp
from jax.experimental import pallas as pl
from jax.experimental.pallas import tpu as pltpu

_BN_EPS = 1e-5


def _pick_tile(n, cands=(2048, 1024, 512, 256, 128, 64, 32, 16, 8)):
    for c in cands:
        if n % c == 0:
            return c
    return n


def _mm_stats_kernel(x_ref, w_ref, y_ref, sum_ref, sumsq_ref):
    x = x_ref[...].astype(jnp.bfloat16)
    y = jnp.dot(x, w_ref[...], preferred_element_type=jnp.float32)
    y_ref[...] = y.astype(jnp.bfloat16)
    y3 = y.reshape(-1, 8, y.shape[-1])
    sum_ref[...] = jnp.sum(y3, axis=0)
    sumsq_ref[...] = jnp.sum(y3 * y3, axis=0)


def _affine_relu_kernel(y_ref, scale_ref, shift_ref, o_ref):
    y = y_ref[...].astype(jnp.float32)
    o_ref[...] = jnp.maximum(y * scale_ref[...] + shift_ref[...], 0.0)


@jax.jit
def _forward(src, w, gamma, beta):
    B, T, Din = src.shape
    E = w.shape[1]
    N = B * T
    x2d = src.reshape(N, Din)
    w_bf = w.astype(jnp.bfloat16)

    tn = _pick_tile(N)
    num_n = N // tn

    y_bf, sum_p, sumsq_p = pl.pallas_call(
        _mm_stats_kernel,
        out_shape=(
            jax.ShapeDtypeStruct((N, E), jnp.bfloat16),
            jax.ShapeDtypeStruct((num_n * 8, E), jnp.float32),
            jax.ShapeDtypeStruct((num_n * 8, E), jnp.float32),
        ),
        grid=(num_n,),
        in_specs=[
            pl.BlockSpec((tn, Din), lambda n: (n, 0)),
            pl.BlockSpec((Din, E), lambda n: (0, 0)),
        ],
        out_specs=[
            pl.BlockSpec((tn, E), lambda n: (n, 0)),
            pl.BlockSpec((8, E), lambda n: (n, 0)),
            pl.BlockSpec((8, E), lambda n: (n, 0)),
        ],
        compiler_params=pltpu.CompilerParams(
            dimension_semantics=("parallel",),
            vmem_limit_bytes=48 * 1024 * 1024,
        ),
    )(x2d, w_bf)

    inv_n = 1.0 / N
    sum_c = jnp.sum(sum_p, axis=0)
    sumsq_c = jnp.sum(sumsq_p, axis=0)
    mean = sum_c * inv_n
    var = jnp.maximum(sumsq_c * inv_n - mean * mean, 0.0)
    inv_std = jax.lax.rsqrt(var + _BN_EPS)
    g = gamma.reshape(-1)
    scale = (g * inv_std).reshape(1, E).astype(jnp.float32)
    shift = (beta.reshape(-1) - mean * g * inv_std).reshape(1, E).astype(jnp.float32)

    tm = _pick_tile(N, cands=(4096, 2048, 1024, 512, 256, 128, 64, 32, 16, 8))
    num_m = N // tm
    out2d = pl.pallas_call(
        _affine_relu_kernel,
        out_shape=jax.ShapeDtypeStruct((N, E), src.dtype),
        grid=(num_m,),
        in_specs=[
            pl.BlockSpec((tm, E), lambda m: (m, 0)),
            pl.BlockSpec((1, E), lambda m: (0, 0)),
            pl.BlockSpec((1, E), lambda m: (0, 0)),
        ],
        out_specs=pl.BlockSpec((tm, E), lambda m: (m, 0)),
        compiler_params=pltpu.CompilerParams(
            dimension_semantics=("parallel",),
            vmem_limit_bytes=48 * 1024 * 1024,
        ),
    )(y_bf, scale, shift)

    return out2d.reshape(B, T, E)


def kernel(src, w, b, gamma, beta):
    del b
    return _forward(src, w, gamma, beta)
